# baseline (speedup 1.0000x reference)
import functools

import jax
import jax.numpy as jnp
from jax import lax
from jax.experimental import pallas as pl
from jax.experimental.pallas import tpu as pltpu
from jax.experimental.pallas import tpu_sc as plsc

B = 16384
D = 32
NC, NS, L = 2, 16, 16
NW = NC * NS
BPW = B // NW
NCH = 4
CHUNK = BPW // NCH
GROUPS = BPW // L
NWROWS = 2 * D + 1


def _body(uid_hbm, iid_hbm, ut_hbm, it_hbm, w_hbm, out_hbm,
          idx_u, idx_i, u_rows, i_rows, w_v, out_v, sem_u, sem_i):
    wid = lax.axis_index("s") * NC + lax.axis_index("c")
    pltpu.sync_copy(uid_hbm.at[wid], idx_u)
    pltpu.sync_copy(iid_hbm.at[wid], idx_i)
    cps = []
    for j in range(NCH):
        cps.append(pltpu.async_copy(ut_hbm.at[idx_u.at[j]], u_rows.at[j], sem_u))
        cps.append(pltpu.async_copy(it_hbm.at[idx_i.at[j]], i_rows.at[j], sem_i))
    pltpu.sync_copy(w_hbm, w_v)
    for cp in cps:
        cp.wait()

    lanes = lax.iota(jnp.int32, L)

    def group(g, carry):
        j = g // (CHUNK // L)
        rows = (g % (CHUNK // L)) * L + lanes
        jvec = jnp.full((L,), j, dtype=jnp.int32)
        acc = w_v[2 * D]
        for d in range(D):
            cols = jnp.full((L,), d, dtype=jnp.int32)
            acc = acc + plsc.load_gather(u_rows, [jvec, rows, cols]) * w_v[d]
            acc = acc + plsc.load_gather(i_rows, [jvec, rows, cols]) * w_v[D + d]
        out_v[pl.ds(g * L, L)] = acc
        return carry

    lax.fori_loop(0, GROUPS, group, 0)
    pltpu.sync_copy(out_v, out_hbm.at[wid])


_mesh = plsc.VectorSubcoreMesh(core_axis_name="c", subcore_axis_name="s")

_ncf = functools.partial(
    pl.kernel, mesh=_mesh,
    compiler_params=pltpu.CompilerParams(
        needs_layout_passes=False, use_tc_tiling_on_sc=False),
    out_type=jax.ShapeDtypeStruct((NW, BPW), jnp.float32),
    scratch_types=[
        pltpu.VMEM((NCH, CHUNK), jnp.int32),
        pltpu.VMEM((NCH, CHUNK), jnp.int32),
        pltpu.VMEM((NCH, CHUNK, D), jnp.float32),
        pltpu.VMEM((NCH, CHUNK, D), jnp.float32),
        pltpu.VMEM((NWROWS, L), jnp.float32),
        pltpu.VMEM((BPW,), jnp.float32),
        pltpu.SemaphoreType.DMA,
        pltpu.SemaphoreType.DMA,
    ],
)(_body)


def kernel(user_ids, item_ids, user_table, item_table, fc_w, fc_b):
    uid = user_ids.astype(jnp.int32).reshape(NW, NCH, CHUNK)
    iid = item_ids.astype(jnp.int32).reshape(NW, NCH, CHUNK)
    w_all = jnp.tile(
        jnp.concatenate([fc_w.reshape(-1), fc_b.reshape(-1)])
        .astype(jnp.float32).reshape(NWROWS, 1),
        (1, L),
    )
    out = _ncf(uid, iid, user_table, item_table, w_all)
    return out.reshape(B, 1)
